# baseline (speedup 1.0000x reference)
import jax
import jax.numpy as jnp
from jax import lax
from jax.experimental import pallas as pl
from jax.experimental.pallas import tpu as pltpu
from jax.experimental.pallas import tpu_sc as plsc

_NC = 2
_NS = 16
_NW = _NC * _NS
_LANES = 16
_UNROLL = 16


def _tc_add(x_ref, row_ref, o_ref):
    o_ref[:, :] = x_ref[:, :] + row_ref[:, :]


def kernel(x, row_embed):
    B, L, D = x.shape
    table = row_embed[:L]
    x2 = x.reshape(B * L, D)

    tc_out = pl.pallas_call(
        _tc_add,
        grid=(B - 1,),
        in_specs=[
            pl.BlockSpec((L, D), lambda b: (b, 0)),
            pl.BlockSpec((L, D), lambda b: (0, 0)),
        ],
        out_specs=pl.BlockSpec((L, D), lambda b: (b, 0)),
        out_shape=jax.ShapeDtypeStruct(((B - 1) * L, D), x.dtype),
        compiler_params=pltpu.CompilerParams(
            dimension_semantics=("parallel",),
        ),
    )(x2, table)

    l_span = L // _NW
    span_w = l_span * D
    x_flat = x.reshape(-1)
    t_flat = table.reshape(-1)
    sc_base = (B - 1) * L * D

    mesh = plsc.VectorSubcoreMesh(core_axis_name="c", subcore_axis_name="s")

    @pl.kernel(
        out_type=jax.ShapeDtypeStruct((L * D,), x.dtype),
        mesh=mesh,
        scratch_types=[
            pltpu.VMEM((span_w,), jnp.float32),
            pltpu.VMEM((span_w,), jnp.float32),
            pltpu.SemaphoreType.DMA,
            pltpu.SemaphoreType.DMA,
        ],
    )
    def sc_kernel(x_hbm, row_hbm, o_hbm, tbuf, obuf, sem_t, sem_x):
        wid = lax.axis_index("s") * _NC + lax.axis_index("c")
        off = wid * span_w
        t_copy = pltpu.async_copy(row_hbm.at[pl.ds(off, span_w)], tbuf, sem_t)
        x_copy = pltpu.async_copy(
            x_hbm.at[pl.ds(sc_base + off, span_w)], obuf, sem_x
        )
        x_copy.wait()
        t_copy.wait()

        @pl.loop(0, span_w, step=_UNROLL * _LANES)
        def _(k):
            for u in range(_UNROLL):
                s = u * _LANES
                v = tbuf[pl.ds(k + s, _LANES)]
                plsc.addupdate(obuf.at[pl.ds(k + s, _LANES)], v)

        pltpu.sync_copy(obuf, o_hbm.at[pl.ds(off, span_w)])

    sc_out = sc_kernel(x_flat, t_flat).reshape(L, D)
    out = jnp.concatenate([tc_out, sc_out], axis=0)
    return out.reshape(B, L, D)


# speedup vs baseline: 4.9564x; 4.9564x over previous
import jax
import jax.numpy as jnp
from jax.experimental import pallas as pl
from jax.experimental.pallas import tpu as pltpu


def _add_kernel(x_ref, row_ref, o_ref):
    o_ref[:, :] = x_ref[:, :] + row_ref[:, :]


def kernel(x, row_embed):
    B, L, D = x.shape
    table = row_embed[:L]
    x2 = x.reshape(B * L, D)
    out = pl.pallas_call(
        _add_kernel,
        grid=(B,),
        in_specs=[
            pl.BlockSpec((L, D), lambda b: (b, 0)),
            pl.BlockSpec((L, D), lambda b: (0, 0)),
        ],
        out_specs=pl.BlockSpec((L, D), lambda b: (b, 0)),
        out_shape=jax.ShapeDtypeStruct((B * L, D), x.dtype),
        compiler_params=pltpu.CompilerParams(
            dimension_semantics=("parallel",),
        ),
    )(x2, table)
    return out.reshape(B, L, D)


# speedup vs baseline: 5.0460x; 1.0181x over previous
import jax
import jax.numpy as jnp
from jax.experimental import pallas as pl
from jax.experimental.pallas import tpu as pltpu

_CH = 512
_NBUF = 4


def _make(B, L, D):
    n_chunks = B * L // _CH
    n_tchunks = L // _CH

    def body(x_hbm, t_hbm, o_hbm, tbuf, xbuf, obuf, sem_t, sem_x, sem_o):
        def t_copy(j):
            return pltpu.make_async_copy(
                t_hbm.at[pl.ds(j * _CH, _CH)],
                tbuf.at[pl.ds(j * _CH, _CH)],
                sem_t.at[j],
            )

        def x_copy(i):
            return pltpu.make_async_copy(
                x_hbm.at[pl.ds(i * _CH, _CH)], xbuf.at[i % _NBUF], sem_x.at[i % _NBUF]
            )

        def o_copy(i):
            return pltpu.make_async_copy(
                obuf.at[i % _NBUF], o_hbm.at[pl.ds(i * _CH, _CH)], sem_o.at[i % _NBUF]
            )

        t_copy(0).start()
        x_copy(0).start()
        for j in range(1, n_tchunks):
            t_copy(j).start()
        for i in range(1, _NBUF):
            x_copy(i).start()

        for i in range(n_chunks):
            s = i % _NBUF
            j = i % n_tchunks
            x_copy(i).wait()
            if i < n_tchunks:
                t_copy(j).wait()
            if i >= _NBUF:
                o_copy(i - _NBUF).wait()
            obuf[s, :, :] = xbuf[s, :, :] + tbuf[pl.ds(j * _CH, _CH), :]
            o_copy(i).start()
            if i + _NBUF < n_chunks:
                x_copy(i + _NBUF).start()
        for i in range(n_chunks - _NBUF, n_chunks):
            o_copy(i).wait()

    return body


def kernel(x, row_embed):
    B, L, D = x.shape
    table = row_embed[:L]
    x2 = x.reshape(B * L, D)
    out = pl.pallas_call(
        _make(B, L, D),
        in_specs=[
            pl.BlockSpec(memory_space=pltpu.MemorySpace.HBM),
            pl.BlockSpec(memory_space=pltpu.MemorySpace.HBM),
        ],
        out_specs=pl.BlockSpec(memory_space=pltpu.MemorySpace.HBM),
        out_shape=jax.ShapeDtypeStruct((B * L, D), x.dtype),
        scratch_shapes=[
            pltpu.VMEM((L, D), jnp.float32),
            pltpu.VMEM((_NBUF, _CH, D), jnp.float32),
            pltpu.VMEM((_NBUF, _CH, D), jnp.float32),
            pltpu.SemaphoreType.DMA((L // _CH,)),
            pltpu.SemaphoreType.DMA((_NBUF,)),
            pltpu.SemaphoreType.DMA((_NBUF,)),
        ],
    )(x2, table)
    return out.reshape(B, L, D)


# speedup vs baseline: 5.0579x; 1.0024x over previous
import jax
import jax.numpy as jnp
from jax.experimental import pallas as pl
from jax.experimental.pallas import tpu as pltpu

_CH = 256
_NBUF = 8


def _make(B, L, D):
    n_chunks = B * L // _CH
    n_tchunks = L // _CH

    def body(x_hbm, t_hbm, o_hbm, tbuf, xbuf, obuf, sem_t, sem_x, sem_o):
        def t_copy(j):
            return pltpu.make_async_copy(
                t_hbm.at[pl.ds(j * _CH, _CH)],
                tbuf.at[pl.ds(j * _CH, _CH)],
                sem_t.at[j],
            )

        def x_copy(i):
            return pltpu.make_async_copy(
                x_hbm.at[pl.ds(i * _CH, _CH)], xbuf.at[i % _NBUF], sem_x.at[i % _NBUF]
            )

        def o_copy(i):
            return pltpu.make_async_copy(
                obuf.at[i % _NBUF], o_hbm.at[pl.ds(i * _CH, _CH)], sem_o.at[i % _NBUF]
            )

        t_copy(0).start()
        x_copy(0).start()
        for j in range(1, n_tchunks):
            t_copy(j).start()
        for i in range(1, _NBUF):
            x_copy(i).start()

        for i in range(n_chunks):
            s = i % _NBUF
            j = i % n_tchunks
            x_copy(i).wait()
            if i < n_tchunks:
                t_copy(j).wait()
            if i >= _NBUF:
                o_copy(i - _NBUF).wait()
            obuf[s, :, :] = xbuf[s, :, :] + tbuf[pl.ds(j * _CH, _CH), :]
            o_copy(i).start()
            if i + _NBUF < n_chunks:
                x_copy(i + _NBUF).start()
        for i in range(n_chunks - _NBUF, n_chunks):
            o_copy(i).wait()

    return body


def kernel(x, row_embed):
    B, L, D = x.shape
    table = row_embed[:L]
    x2 = x.reshape(B * L, D)
    out = pl.pallas_call(
        _make(B, L, D),
        in_specs=[
            pl.BlockSpec(memory_space=pltpu.MemorySpace.HBM),
            pl.BlockSpec(memory_space=pltpu.MemorySpace.HBM),
        ],
        out_specs=pl.BlockSpec(memory_space=pltpu.MemorySpace.HBM),
        out_shape=jax.ShapeDtypeStruct((B * L, D), x.dtype),
        scratch_shapes=[
            pltpu.VMEM((L, D), jnp.float32),
            pltpu.VMEM((_NBUF, _CH, D), jnp.float32),
            pltpu.VMEM((_NBUF, _CH, D), jnp.float32),
            pltpu.SemaphoreType.DMA((L // _CH,)),
            pltpu.SemaphoreType.DMA((_NBUF,)),
            pltpu.SemaphoreType.DMA((_NBUF,)),
        ],
    )(x2, table)
    return out.reshape(B, L, D)


# speedup vs baseline: 5.3999x; 1.0676x over previous
import jax
import jax.numpy as jnp
from jax.experimental import pallas as pl
from jax.experimental.pallas import tpu as pltpu

_CH = 1024
_NBUF = 4


def _make(B, L, D):
    n_chunks = B * L // _CH
    n_tchunks = L // _CH

    def body(x_hbm, t_hbm, o_hbm, tbuf, xbuf, obuf, sem_t, sem_x, sem_o):
        def t_copy(j):
            return pltpu.make_async_copy(
                t_hbm.at[pl.ds(j * _CH, _CH)],
                tbuf.at[pl.ds(j * _CH, _CH)],
                sem_t.at[j],
            )

        def x_copy(i):
            return pltpu.make_async_copy(
                x_hbm.at[pl.ds(i * _CH, _CH)], xbuf.at[i % _NBUF], sem_x.at[i % _NBUF]
            )

        def o_copy(i):
            return pltpu.make_async_copy(
                obuf.at[i % _NBUF], o_hbm.at[pl.ds(i * _CH, _CH)], sem_o.at[i % _NBUF]
            )

        t_copy(0).start()
        x_copy(0).start()
        for j in range(1, n_tchunks):
            t_copy(j).start()
        for i in range(1, _NBUF):
            x_copy(i).start()

        for i in range(n_chunks):
            s = i % _NBUF
            j = i % n_tchunks
            x_copy(i).wait()
            if i < n_tchunks:
                t_copy(j).wait()
            if i >= _NBUF:
                o_copy(i - _NBUF).wait()
            obuf[s, :, :] = xbuf[s, :, :] + tbuf[pl.ds(j * _CH, _CH), :]
            o_copy(i).start()
            if i + _NBUF < n_chunks:
                x_copy(i + _NBUF).start()
        for i in range(n_chunks - _NBUF, n_chunks):
            o_copy(i).wait()

    return body


def kernel(x, row_embed):
    B, L, D = x.shape
    table = row_embed[:L]
    x2 = x.reshape(B * L, D)
    out = pl.pallas_call(
        _make(B, L, D),
        in_specs=[
            pl.BlockSpec(memory_space=pltpu.MemorySpace.HBM),
            pl.BlockSpec(memory_space=pltpu.MemorySpace.HBM),
        ],
        out_specs=pl.BlockSpec(memory_space=pltpu.MemorySpace.HBM),
        out_shape=jax.ShapeDtypeStruct((B * L, D), x.dtype),
        scratch_shapes=[
            pltpu.VMEM((L, D), jnp.float32),
            pltpu.VMEM((_NBUF, _CH, D), jnp.float32),
            pltpu.VMEM((_NBUF, _CH, D), jnp.float32),
            pltpu.SemaphoreType.DMA((L // _CH,)),
            pltpu.SemaphoreType.DMA((_NBUF,)),
            pltpu.SemaphoreType.DMA((_NBUF,)),
        ],
    )(x2, table)
    return out.reshape(B, L, D)


# speedup vs baseline: 5.4497x; 1.0092x over previous
import jax
import jax.numpy as jnp
from jax.experimental import pallas as pl
from jax.experimental.pallas import tpu as pltpu

_CH = 2048
_NBUF = 3


def _make(B, L, D):
    n_chunks = B * L // _CH
    n_tchunks = L // _CH

    def body(x_hbm, t_hbm, o_hbm, tbuf, xbuf, obuf, sem_t, sem_x, sem_o):
        def t_copy(j):
            return pltpu.make_async_copy(
                t_hbm.at[pl.ds(j * _CH, _CH)],
                tbuf.at[pl.ds(j * _CH, _CH)],
                sem_t.at[j],
            )

        def x_copy(i):
            return pltpu.make_async_copy(
                x_hbm.at[pl.ds(i * _CH, _CH)], xbuf.at[i % _NBUF], sem_x.at[i % _NBUF]
            )

        def o_copy(i):
            return pltpu.make_async_copy(
                obuf.at[i % _NBUF], o_hbm.at[pl.ds(i * _CH, _CH)], sem_o.at[i % _NBUF]
            )

        t_copy(0).start()
        x_copy(0).start()
        for j in range(1, n_tchunks):
            t_copy(j).start()
        for i in range(1, _NBUF):
            x_copy(i).start()

        for i in range(n_chunks):
            s = i % _NBUF
            j = i % n_tchunks
            x_copy(i).wait()
            if i < n_tchunks:
                t_copy(j).wait()
            if i >= _NBUF:
                o_copy(i - _NBUF).wait()
            obuf[s, :, :] = xbuf[s, :, :] + tbuf[pl.ds(j * _CH, _CH), :]
            o_copy(i).start()
            if i + _NBUF < n_chunks:
                x_copy(i + _NBUF).start()
        for i in range(n_chunks - _NBUF, n_chunks):
            o_copy(i).wait()

    return body


def kernel(x, row_embed):
    B, L, D = x.shape
    table = row_embed[:L]
    x2 = x.reshape(B * L, D)
    out = pl.pallas_call(
        _make(B, L, D),
        in_specs=[
            pl.BlockSpec(memory_space=pltpu.MemorySpace.HBM),
            pl.BlockSpec(memory_space=pltpu.MemorySpace.HBM),
        ],
        out_specs=pl.BlockSpec(memory_space=pltpu.MemorySpace.HBM),
        out_shape=jax.ShapeDtypeStruct((B * L, D), x.dtype),
        scratch_shapes=[
            pltpu.VMEM((L, D), jnp.float32),
            pltpu.VMEM((_NBUF, _CH, D), jnp.float32),
            pltpu.VMEM((_NBUF, _CH, D), jnp.float32),
            pltpu.SemaphoreType.DMA((L // _CH,)),
            pltpu.SemaphoreType.DMA((_NBUF,)),
            pltpu.SemaphoreType.DMA((_NBUF,)),
        ],
    )(x2, table)
    return out.reshape(B, L, D)


# speedup vs baseline: 5.4711x; 1.0039x over previous
import jax
import jax.numpy as jnp
from jax.experimental import pallas as pl
from jax.experimental.pallas import tpu as pltpu

_CH = 1024
_NBUF = 6


def _make(B, L, D):
    n_chunks = B * L // _CH
    n_tchunks = L // _CH

    def body(x_hbm, t_hbm, o_hbm, tbuf, xbuf, obuf, sem_t, sem_x, sem_o):
        def t_copy(j):
            return pltpu.make_async_copy(
                t_hbm.at[pl.ds(j * _CH, _CH)],
                tbuf.at[pl.ds(j * _CH, _CH)],
                sem_t.at[j],
            )

        def x_copy(i):
            return pltpu.make_async_copy(
                x_hbm.at[pl.ds(i * _CH, _CH)], xbuf.at[i % _NBUF], sem_x.at[i % _NBUF]
            )

        def o_copy(i):
            return pltpu.make_async_copy(
                obuf.at[i % _NBUF], o_hbm.at[pl.ds(i * _CH, _CH)], sem_o.at[i % _NBUF]
            )

        t_copy(0).start()
        x_copy(0).start()
        for j in range(1, n_tchunks):
            t_copy(j).start()
        for i in range(1, _NBUF):
            x_copy(i).start()

        for i in range(n_chunks):
            s = i % _NBUF
            j = i % n_tchunks
            x_copy(i).wait()
            if i < n_tchunks:
                t_copy(j).wait()
            if i >= _NBUF:
                o_copy(i - _NBUF).wait()
            obuf[s, :, :] = xbuf[s, :, :] + tbuf[pl.ds(j * _CH, _CH), :]
            o_copy(i).start()
            if i + _NBUF < n_chunks:
                x_copy(i + _NBUF).start()
        for i in range(n_chunks - _NBUF, n_chunks):
            o_copy(i).wait()

    return body


def kernel(x, row_embed):
    B, L, D = x.shape
    table = row_embed[:L]
    x2 = x.reshape(B * L, D)
    out = pl.pallas_call(
        _make(B, L, D),
        in_specs=[
            pl.BlockSpec(memory_space=pltpu.MemorySpace.HBM),
            pl.BlockSpec(memory_space=pltpu.MemorySpace.HBM),
        ],
        out_specs=pl.BlockSpec(memory_space=pltpu.MemorySpace.HBM),
        out_shape=jax.ShapeDtypeStruct((B * L, D), x.dtype),
        scratch_shapes=[
            pltpu.VMEM((L, D), jnp.float32),
            pltpu.VMEM((_NBUF, _CH, D), jnp.float32),
            pltpu.VMEM((_NBUF, _CH, D), jnp.float32),
            pltpu.SemaphoreType.DMA((L // _CH,)),
            pltpu.SemaphoreType.DMA((_NBUF,)),
            pltpu.SemaphoreType.DMA((_NBUF,)),
        ],
    )(x2, table)
    return out.reshape(B, L, D)
